# baseline (speedup 1.0000x reference)
import jax
import jax.numpy as jnp
from jax import lax
from jax.experimental import pallas as pl
from jax.experimental.pallas import tpu as pltpu, tpu_sc as plsc

N = 10000
E = 320000
D = 128
NT = 16
NCORE = 2
EPC = E // NCORE
EPT = EPC // NT
K = 80
CPB = 25
BE = CPB * K
NBLK = EPT // BE
NP = 10240
RPT = NP // NT
BN = 1000


def _fin_body(zp_ref, sp_ref, w_ref, b_ref, out_ref):
    zsum = zp_ref[0] + zp_ref[1]
    ssum = sp_ref[0] + sp_ref[1]
    out_ref[...] = (
        lax.dot_general(
            zsum, w_ref[...],
            (((1,), (1,)), ((), ())),
            preferred_element_type=jnp.float32,
        )
        + ssum * b_ref[...]
    )


def _finish(zp, sp, W, b):
    return pl.pallas_call(
        _fin_body,
        grid=(N // BN,),
        in_specs=[
            pl.BlockSpec((NCORE, BN, D), lambda i: (0, i, 0)),
            pl.BlockSpec((NCORE, BN, 1), lambda i: (0, i, 0)),
            pl.BlockSpec((D, D), lambda i: (0, 0)),
            pl.BlockSpec((1, D), lambda i: (0, 0)),
        ],
        out_specs=pl.BlockSpec((BN, D), lambda i: (i, 0)),
        out_shape=jax.ShapeDtypeStruct((N, D), jnp.float32),
    )(zp, sp.reshape(NCORE, NP, 1), W, b.reshape(1, D))


def _sc_body(x, src, dst3, dstf, w, zp_out, sp_out,
             acc, acc_s, src_s, w_s, dst_s, dst_f, zb, gb0, gb1, gb2,
             gsem0, gsem1, gsem2, ssem0, ssem1, ssem2, wsem):
    c = lax.axis_index("c")
    t = lax.axis_index("s")
    bufs = ((gb0, gsem0, ssem0), (gb1, gsem1, ssem1), (gb2, gsem2, ssem2))

    def zrow(i, _):
        for j in range(D // 16):
            gb0[i, pl.ds(j * 16, 16)] = jnp.zeros((16,), jnp.float32)
        return 0
    lax.fori_loop(0, K, zrow, 0, unroll=4)
    def zs(i, _):
        zb[pl.ds(i * 16, 16)] = jnp.zeros((16,), jnp.float32)
        return 0
    lax.fori_loop(0, RPT // 16, zs, 0, unroll=4)
    for hh in range(RPT // K):
        pltpu.sync_copy(gb0, acc.at[pl.ds(t * RPT + hh * K, K)])
    if RPT % K:
        pltpu.sync_copy(
            gb0.at[pl.ds(0, RPT % K)],
            acc.at[pl.ds(t * RPT + (RPT // K) * K, RPT % K)],
        )
    pltpu.sync_copy(zb, acc_s.at[pl.ds(t * RPT, RPT)])
    plsc.subcore_barrier()

    ebase = c * EPC + t * EPT
    bbase = (c * EPC + t * EPT) // BE

    def start(j, b):
        gb, gsem, _ = bufs[b]
        pltpu.async_copy(x.at[src_s.at[pl.ds(j * K, K)]], gb, gsem)

    def wait_g(b):
        gb, gsem, _ = bufs[b]
        pltpu.make_async_copy(x.at[src_s.at[pl.ds(0, K)]], gb, gsem).wait()

    def scale(j, b):
        gb, _, _ = bufs[b]
        woff = j * K
        def body(g, _):
            e0 = g * 16
            wv = w_s[pl.ds(woff + e0, 16)]
            for jj in range(16):
                ws = wv[jj]
                for q in range(D // 16):
                    sl = pl.ds(q * 16, 16)
                    gb[e0 + jj, sl] = gb[e0 + jj, sl] * ws
            return 0
        lax.fori_loop(0, K // 16, body, 0)
        return 0

    def scat(j, b):
        gb, _, ssem = bufs[b]
        pltpu.async_copy(gb, acc.at[dst_s.at[0, j]], ssem, add=True)

    def wait_s(j, b):
        gb, _, ssem = bufs[b]
        pltpu.make_async_copy(gb, acc.at[dst_s.at[0, j]], ssem).wait()

    def block(B, _):
        base = ebase + B * BE
        pltpu.sync_copy(src.at[pl.ds(base, BE)], src_s)
        pltpu.sync_copy(w.at[pl.ds(base, BE)], w_s)
        pltpu.sync_copy(dst3.at[pl.ds(bbase + B, 1)], dst_s)
        pltpu.sync_copy(dstf.at[pl.ds(base, BE)], dst_f)
        pltpu.async_copy(w_s, acc_s.at[dst_f], wsem, add=True)

        start(0, 0)
        start(1, 1)
        for j in range(CPB):
            bb = j % 3
            wait_g(bb); scale(j, bb); scat(j, bb)
            if j == 0:
                start(2, 2)
            elif j + 2 < CPB:
                wait_s(j - 1, (j - 1) % 3); start(j + 2, (j + 2) % 3)
        wait_s(CPB - 3, (CPB - 3) % 3)
        wait_s(CPB - 2, (CPB - 2) % 3)
        wait_s(CPB - 1, (CPB - 1) % 3)
        pltpu.make_async_copy(w_s, acc_s.at[dst_f], wsem).wait()
        return 0

    lax.fori_loop(0, NBLK, block, 0)
    plsc.subcore_barrier()

    pltpu.sync_copy(
        acc.at[pl.ds(t * RPT, RPT)],
        zp_out.at[c, pl.ds(t * RPT, RPT)],
    )
    pltpu.sync_copy(
        acc_s.at[pl.ds(t * RPT, RPT)],
        sp_out.at[pl.ds(c * NP + t * RPT, RPT)],
    )


@jax.jit
def kernel(x, edge_index, w, W, b):
    src = edge_index[0]
    dstf = edge_index[1]
    dst3 = edge_index[1].reshape(E // BE, CPB, K)
    mesh = plsc.VectorSubcoreMesh(core_axis_name="c", subcore_axis_name="s")
    sc = pl.kernel(
        _sc_body,
        out_type=(
            jax.ShapeDtypeStruct((NCORE, NP, D), jnp.float32),
            jax.ShapeDtypeStruct((NCORE * NP,), jnp.float32),
        ),
        mesh=mesh,
        scratch_types=[
            pltpu.VMEM_SHARED((NP, D), jnp.float32),
            pltpu.VMEM_SHARED((NP,), jnp.float32),
            pltpu.VMEM((BE,), jnp.int32),
            pltpu.VMEM((BE,), jnp.float32),
            pltpu.VMEM((1, CPB, K), jnp.int32),
            pltpu.VMEM((BE,), jnp.int32),
            pltpu.VMEM((RPT,), jnp.float32),
            pltpu.VMEM((K, D), jnp.float32),
            pltpu.VMEM((K, D), jnp.float32),
            pltpu.VMEM((K, D), jnp.float32),
            pltpu.SemaphoreType.DMA,
            pltpu.SemaphoreType.DMA,
            pltpu.SemaphoreType.DMA,
            pltpu.SemaphoreType.DMA,
            pltpu.SemaphoreType.DMA,
            pltpu.SemaphoreType.DMA,
            pltpu.SemaphoreType.DMA,
        ],
    )
    zp, sp = sc(x, src, dst3, dstf, w)
    return _finish(zp, sp, W, b)


# speedup vs baseline: 1.0542x; 1.0542x over previous
import jax
import jax.numpy as jnp
from jax import lax
from jax.experimental import pallas as pl
from jax.experimental.pallas import tpu as pltpu, tpu_sc as plsc

N = 10000
E = 320000
D = 128
NT = 16
NCORE = 2
EPC = E // NCORE
EPT = EPC // NT
K = 80
CPB = 25
BE = CPB * K
NBLK = EPT // BE
NP = 10240
RPT = NP // NT
BN = 1000


def _matmul_body(x_ref, w_ref, b_ref, out_ref):
    out_ref[...] = (
        lax.dot_general(
            x_ref[...], w_ref[...],
            (((1,), (1,)), ((), ())),
            preferred_element_type=jnp.float32,
        )
        + b_ref[...]
    )


def _compute_h(x, W, b):
    return pl.pallas_call(
        _matmul_body,
        grid=(N // BN,),
        in_specs=[
            pl.BlockSpec((BN, D), lambda i: (i, 0)),
            pl.BlockSpec((D, D), lambda i: (0, 0)),
            pl.BlockSpec((1, D), lambda i: (0, 0)),
        ],
        out_specs=pl.BlockSpec((BN, D), lambda i: (i, 0)),
        out_shape=jax.ShapeDtypeStruct((N, D), jnp.float32),
    )(x, W, b.reshape(1, D))


def _add_body(p_ref, out_ref):
    out_ref[...] = p_ref[0] + p_ref[1]


def _combine(partials):
    return pl.pallas_call(
        _add_body,
        grid=(N // BN,),
        in_specs=[pl.BlockSpec((NCORE, BN, D), lambda i: (0, i, 0))],
        out_specs=pl.BlockSpec((BN, D), lambda i: (i, 0)),
        out_shape=jax.ShapeDtypeStruct((N, D), jnp.float32),
    )(partials)


def _sc_body(h, src, dst3, w, out, acc,
             src_s, w_s, dst_s, gb0, gb1, gb2,
             gsem0, gsem1, gsem2, ssem0, ssem1, ssem2):
    c = lax.axis_index("c")
    t = lax.axis_index("s")
    bufs = ((gb0, gsem0, ssem0), (gb1, gsem1, ssem1), (gb2, gsem2, ssem2))

    def zrow(i, _):
        for j in range(D // 16):
            gb0[i, pl.ds(j * 16, 16)] = jnp.zeros((16,), jnp.float32)
        return 0
    lax.fori_loop(0, K, zrow, 0, unroll=4)
    for hh in range(RPT // K):
        pltpu.sync_copy(gb0, acc.at[pl.ds(t * RPT + hh * K, K)])
    if RPT % K:
        pltpu.sync_copy(
            gb0.at[pl.ds(0, RPT % K)],
            acc.at[pl.ds(t * RPT + (RPT // K) * K, RPT % K)],
        )
    plsc.subcore_barrier()

    ebase = c * EPC + t * EPT
    bbase = (c * EPC + t * EPT) // BE

    def start(j, b):
        gb, gsem, _ = bufs[b]
        pltpu.async_copy(h.at[src_s.at[pl.ds(j * K, K)]], gb, gsem)

    def wait_g(b):
        gb, gsem, _ = bufs[b]
        pltpu.make_async_copy(h.at[src_s.at[pl.ds(0, K)]], gb, gsem).wait()

    def scale(j, b):
        gb, _, _ = bufs[b]
        woff = j * K
        def body(g, _):
            e0 = g * 16
            wv = w_s[pl.ds(woff + e0, 16)]
            for jj in range(16):
                ws = wv[jj]
                for q in range(D // 16):
                    sl = pl.ds(q * 16, 16)
                    gb[e0 + jj, sl] = gb[e0 + jj, sl] * ws
            return 0
        lax.fori_loop(0, K // 16, body, 0)
        return 0

    def scat(j, b):
        gb, _, ssem = bufs[b]
        pltpu.async_copy(gb, acc.at[dst_s.at[0, j]], ssem, add=True)

    def wait_s(j, b):
        gb, _, ssem = bufs[b]
        pltpu.make_async_copy(gb, acc.at[dst_s.at[0, j]], ssem).wait()

    def block(B, _):
        base = ebase + B * BE
        pltpu.sync_copy(src.at[pl.ds(base, BE)], src_s)
        pltpu.sync_copy(w.at[pl.ds(base, BE)], w_s)
        pltpu.sync_copy(dst3.at[pl.ds(bbase + B, 1)], dst_s)

        start(0, 0)
        start(1, 1)
        for j in range(CPB):
            bb = j % 3
            wait_g(bb); scale(j, bb); scat(j, bb)
            if j == 0:
                start(2, 2)
            elif j + 2 < CPB:
                wait_s(j - 1, (j - 1) % 3); start(j + 2, (j + 2) % 3)
        wait_s(CPB - 3, (CPB - 3) % 3)
        wait_s(CPB - 2, (CPB - 2) % 3)
        wait_s(CPB - 1, (CPB - 1) % 3)
        return 0

    lax.fori_loop(0, NBLK, block, 0)
    plsc.subcore_barrier()

    pltpu.sync_copy(
        acc.at[pl.ds(t * RPT, RPT)],
        out.at[c, pl.ds(t * RPT, RPT)],
    )


@jax.jit
def kernel(x, edge_index, w, W, b):
    h = _compute_h(x, W, b)
    src = edge_index[0]
    dst3 = edge_index[1].reshape(E // BE, CPB, K)
    mesh = plsc.VectorSubcoreMesh(core_axis_name="c", subcore_axis_name="s")
    sc = pl.kernel(
        _sc_body,
        out_type=jax.ShapeDtypeStruct((NCORE, NP, D), jnp.float32),
        mesh=mesh,
        scratch_types=[
            pltpu.VMEM_SHARED((NP, D), jnp.float32),
            pltpu.VMEM((BE,), jnp.int32),
            pltpu.VMEM((BE,), jnp.float32),
            pltpu.VMEM((1, CPB, K), jnp.int32),
            pltpu.VMEM((K, D), jnp.float32),
            pltpu.VMEM((K, D), jnp.float32),
            pltpu.VMEM((K, D), jnp.float32),
            pltpu.SemaphoreType.DMA,
            pltpu.SemaphoreType.DMA,
            pltpu.SemaphoreType.DMA,
            pltpu.SemaphoreType.DMA,
            pltpu.SemaphoreType.DMA,
            pltpu.SemaphoreType.DMA,
        ],
    )
    partials = sc(h, src, dst3, w)
    return _combine(partials)
